# baseline (speedup 1.0000x reference)
import functools
import jax
import jax.numpy as jnp
from jax import lax
from jax.experimental import pallas as pl
from jax.experimental.pallas import tpu as pltpu
from jax.experimental.pallas import tpu_sc as plsc

D = 64
NC = 2
NS = 16
NW = NC * NS
K = 4
C = K * 128
NBUF = 2

TCW = 512


def _tr_body(x_ref, o_ref):
    t = x_ref[...].T
    o_ref[...] = jnp.concatenate([t[: TCW // 2], t[TCW // 2 :]], axis=1)


def _weight_rows(weight_t):
    V = weight_t.shape[1]
    grid = (V + TCW - 1) // TCW
    w2 = pl.pallas_call(
        _tr_body,
        grid=(grid,),
        in_specs=[pl.BlockSpec((D, TCW), lambda j: (0, j))],
        out_specs=pl.BlockSpec((TCW // 2, 128), lambda j: (j, 0)),
        out_shape=jax.ShapeDtypeStruct((grid * (TCW // 2), 128), jnp.float32),
    )(weight_t)
    return w2.reshape(grid * TCW, D)


def _emb_body(idx_hbm, table_hbm, out_hbm, idx_v, rows_v, gsem0, gsem1,
              osem0, osem1):
    wid = lax.axis_index("s") * NC + lax.axis_index("c")
    b_per_w = out_hbm.shape[0] // NW
    nchunk = b_per_w // C
    idx_rows = b_per_w // 128
    gsems = [gsem0, gsem1]
    osems = [osem0, osem1]

    pltpu.sync_copy(idx_hbm.at[pl.ds(wid * idx_rows, idx_rows)], idx_v)

    def gather_cp(ci, b, j):
        return pltpu.make_async_copy(
            table_hbm.at[idx_v.at[ci * K + j]],
            rows_v.at[b].at[pl.ds(j * 128, 128)],
            gsems[b],
        )

    def out_cp(ci, b):
        return pltpu.make_async_copy(
            rows_v.at[b],
            out_hbm.at[pl.ds(wid * b_per_w + ci * C, C)],
            osems[b],
        )

    def fire_gather(ci, b):
        for j in range(K):
            gather_cp(ci, b, j).start()

    def wait_gather(ci, b):
        for j in range(K):
            gather_cp(ci, b, j).wait()

    fire_gather(0, 0)
    fire_gather(1, 1)

    def step(g, carry):
        for b in range(NBUF):
            ci = NBUF * g + b
            wait_gather(ci, b)
            out_cp(ci, b).start()
        for b in range(NBUF):
            ci = NBUF * g + b
            out_cp(ci, b).wait()

            @pl.when(g < nchunk // NBUF - 1)
            def _():
                fire_gather(ci + NBUF, b)

        return carry

    lax.fori_loop(0, nchunk // NBUF, step, 0)


def kernel(input, weight):
    B = input.shape[0] * input.shape[1]
    v = input.reshape(B // 128, 128).astype(jnp.int32)
    r = v & (TCW - 1)
    idx = v - r + 2 * r - jnp.where(r < TCW // 2, 0, TCW - 1)
    table = _weight_rows(weight.T)

    gather = functools.partial(
        pl.kernel,
        mesh=plsc.VectorSubcoreMesh(core_axis_name="c", subcore_axis_name="s"),
        out_type=jax.ShapeDtypeStruct((B, D), jnp.float32),
        scratch_types=[
            pltpu.VMEM((B // 128 // NW, 128), jnp.int32),
            pltpu.VMEM((NBUF, C, D), jnp.float32),
            pltpu.SemaphoreType.DMA,
            pltpu.SemaphoreType.DMA,
            pltpu.SemaphoreType.DMA,
            pltpu.SemaphoreType.DMA,
        ],
        compiler_params=pltpu.CompilerParams(use_tc_tiling_on_sc=False),
    )(_emb_body)

    out = gather(idx, table)
    return out.reshape(input.shape[0], input.shape[1], D)


# speedup vs baseline: 2.0080x; 2.0080x over previous
import functools
import jax
import jax.numpy as jnp
from jax import lax
from jax.experimental import pallas as pl
from jax.experimental.pallas import tpu as pltpu
from jax.experimental.pallas import tpu_sc as plsc

D = 64
F = 26
NC = 2
NS = 16
NW = NC * NS
NB = 16
NBUF = 2


def _emb_body(idx_hbm, table_hbm, out_hbm, idx_v, rows_v, gsem0, gsem1,
              osem0, osem1):
    wid = lax.axis_index("s") * NC + lax.axis_index("c")
    b_per_w = out_hbm.shape[0] // NW
    nchunk = b_per_w // NB
    gsems = [gsem0, gsem1]
    osems = [osem0, osem1]

    def gather_cp(b0, buf, j):
        return pltpu.make_async_copy(
            table_hbm.at[idx_v.at[buf].at[j]],
            rows_v.at[buf].at[j],
            gsems[buf],
        )

    def out_cp(b0, buf):
        return pltpu.make_async_copy(
            rows_v.at[buf],
            out_hbm.at[pl.ds(b0, NB), pl.ds(0, F), pl.ds(0, D)],
            osems[buf],
        )

    def fire_gather(b0, buf):
        pltpu.sync_copy(idx_hbm.at[pl.ds(b0, NB)], idx_v.at[buf])
        for j in range(NB):
            gather_cp(b0, buf, j).start()

    def wait_gather(b0, buf):
        for j in range(NB):
            gather_cp(b0, buf, j).wait()

    base = wid * b_per_w
    fire_gather(base, 0)
    fire_gather(base + NB, 1)

    def step(g, carry):
        for buf in range(NBUF):
            b0 = base + (NBUF * g + buf) * NB
            wait_gather(b0, buf)
            out_cp(b0, buf).start()
        for buf in range(NBUF):
            b0 = base + (NBUF * g + buf) * NB
            out_cp(b0, buf).wait()

            @pl.when(g < nchunk // NBUF - 1)
            def _():
                fire_gather(b0 + NBUF * NB, buf)

        return carry

    lax.fori_loop(0, nchunk // NBUF, step, 0)


def kernel(input, weight):
    B, F_ = input.shape
    idx = input.astype(jnp.int32)

    gather = functools.partial(
        pl.kernel,
        mesh=plsc.VectorSubcoreMesh(core_axis_name="c", subcore_axis_name="s"),
        out_type=jax.ShapeDtypeStruct((B, 32, 128), jnp.float32),
        scratch_types=[
            pltpu.VMEM((NBUF, NB, F_), jnp.int32),
            pltpu.VMEM((NBUF, NB, F_, D), jnp.float32),
            pltpu.SemaphoreType.DMA,
            pltpu.SemaphoreType.DMA,
            pltpu.SemaphoreType.DMA,
            pltpu.SemaphoreType.DMA,
        ],
        compiler_params=pltpu.CompilerParams(use_tc_tiling_on_sc=False),
    )(_emb_body)

    z = gather(idx, weight)
    return z[:, :F_, :D]


# speedup vs baseline: 2.0103x; 1.0012x over previous
import functools
import jax
import jax.numpy as jnp
from jax import lax
from jax.experimental import pallas as pl
from jax.experimental.pallas import tpu as pltpu
from jax.experimental.pallas import tpu_sc as plsc

D = 64
F = 26
NC = 2
NS = 16
NW = NC * NS
NB = 32
NBUF = 2


def _emb_body(idx_hbm, table_hbm, out_hbm, idx_v, rows_v, gsem0, gsem1,
              osem0, osem1):
    wid = lax.axis_index("s") * NC + lax.axis_index("c")
    b_per_w = out_hbm.shape[0] // NW
    nchunk = b_per_w // NB
    gsems = [gsem0, gsem1]
    osems = [osem0, osem1]

    def gather_cp(b0, buf, j):
        return pltpu.make_async_copy(
            table_hbm.at[idx_v.at[buf].at[j]],
            rows_v.at[buf].at[j],
            gsems[buf],
        )

    def out_cp(b0, buf):
        return pltpu.make_async_copy(
            rows_v.at[buf],
            out_hbm.at[pl.ds(b0, NB), pl.ds(0, F), pl.ds(0, D)],
            osems[buf],
        )

    def fire_gather(b0, buf):
        pltpu.sync_copy(idx_hbm.at[pl.ds(b0, NB)], idx_v.at[buf])
        for j in range(NB):
            gather_cp(b0, buf, j).start()

    def wait_gather(b0, buf):
        for j in range(NB):
            gather_cp(b0, buf, j).wait()

    base = wid * b_per_w
    fire_gather(base, 0)
    fire_gather(base + NB, 1)

    def step(g, carry):
        for buf in range(NBUF):
            b0 = base + (NBUF * g + buf) * NB
            wait_gather(b0, buf)
            out_cp(b0, buf).start()
        for buf in range(NBUF):
            b0 = base + (NBUF * g + buf) * NB
            out_cp(b0, buf).wait()

            @pl.when(g < nchunk // NBUF - 1)
            def _():
                fire_gather(b0 + NBUF * NB, buf)

        return carry

    lax.fori_loop(0, nchunk // NBUF, step, 0)


def kernel(input, weight):
    B, F_ = input.shape
    idx = input.astype(jnp.int32)

    gather = functools.partial(
        pl.kernel,
        mesh=plsc.VectorSubcoreMesh(core_axis_name="c", subcore_axis_name="s"),
        out_type=jax.ShapeDtypeStruct((B, 32, 128), jnp.float32),
        scratch_types=[
            pltpu.VMEM((NBUF, NB, F_), jnp.int32),
            pltpu.VMEM((NBUF, NB, F_, D), jnp.float32),
            pltpu.SemaphoreType.DMA,
            pltpu.SemaphoreType.DMA,
            pltpu.SemaphoreType.DMA,
            pltpu.SemaphoreType.DMA,
        ],
        compiler_params=pltpu.CompilerParams(use_tc_tiling_on_sc=False),
    )(_emb_body)

    z = gather(idx, weight)
    return z[:, :F_, :D]
